# baseline (speedup 1.0000x reference)
import functools

import jax
import jax.numpy as jnp
from jax import lax
from jax.experimental import pallas as pl
from jax.experimental.pallas import tpu as pltpu
from jax.experimental.pallas import tpu_sc as plsc

_NUM_STEPS = 50
_NUM_ENVS = 4096
_FEAT = 128
_NUM_CORES = 2
_EPC = _NUM_ENVS // _NUM_CORES

_NSEM = 8


def _scs_body(data_hbm, fresh_hbm, out_hbm, *sems):
    cid = lax.axis_index("c")
    e0 = cid * _EPC

    def src(s):
        if s == 0:
            return fresh_hbm.at[pl.ds(e0, _EPC), :]
        return data_hbm.at[s - 1, pl.ds(e0, _EPC), :]

    def dst(s):
        return out_hbm.at[pl.ds(e0, _EPC), pl.ds(s * _FEAT, _FEAT)]

    cps = [None] * _NUM_STEPS
    for s in range(_NUM_STEPS):
        if s >= _NSEM:
            cps[s - _NSEM].wait()
        cps[s] = pltpu.async_copy(src(s), dst(s), sems[s % _NSEM])
    for s in range(_NUM_STEPS - _NSEM, _NUM_STEPS):
        cps[s].wait()


def kernel(data, fresh_data):
    mesh = plsc.ScalarSubcoreMesh(axis_name="c")
    run = pl.kernel(
        _scs_body,
        out_type=jax.ShapeDtypeStruct((_NUM_ENVS, _NUM_STEPS * _FEAT), jnp.float32),
        mesh=mesh,
        scratch_types=[pltpu.SemaphoreType.DMA for _ in range(_NSEM)],
    )
    return run(data, fresh_data)


# speedup vs baseline: 34.9249x; 34.9249x over previous
import functools

import jax
import jax.numpy as jnp
from jax import lax
from jax.experimental import pallas as pl
from jax.experimental.pallas import tpu as pltpu
from jax.experimental.pallas import tpu_sc as plsc
from jax._src.pallas import mpmd

_NUM_STEPS = 50
_NUM_ENVS = 4096
_FEAT = 128
_NUM_CORES = 2
_NUM_TILES = 32

_ENVS_SCS = 2048
_ENVS_TEC = _NUM_ENVS - _ENVS_SCS
_EPS = _ENVS_SCS // _NUM_CORES
_EPT = _ENVS_TEC // _NUM_TILES

_NBUF = 4
_DEPTH = 2


def _ring_copy(data_hbm, fresh_hbm, out_hbm, e0, epw, bufs, isems, osems):
    def src(s):
        if s == 0:
            return fresh_hbm.at[pl.ds(e0, epw), :]
        return data_hbm.at[s - 1, pl.ds(e0, epw), :]

    def dst(s):
        return out_hbm.at[pl.ds(e0, epw), pl.ds(s * _FEAT, _FEAT)]

    inc = [None] * _NUM_STEPS
    outc = [None] * _NUM_STEPS
    for s in range(_DEPTH):
        inc[s] = pltpu.async_copy(src(s), bufs[s % _NBUF], isems[s % _NBUF])
    for s in range(_NUM_STEPS):
        b = s % _NBUF
        inc[s].wait()
        outc[s] = pltpu.async_copy(bufs[b], dst(s), osems[b])
        ns = s + _DEPTH
        if ns < _NUM_STEPS:
            if ns >= _NBUF:
                outc[ns - _NBUF].wait()
            inc[ns] = pltpu.async_copy(src(ns), bufs[ns % _NBUF], isems[ns % _NBUF])
    for s in range(_NUM_STEPS - _NBUF, _NUM_STEPS):
        outc[s].wait()


def _scs_body(data_hbm, fresh_hbm, out_hbm, *scratch):
    sbufs = scratch[:_NBUF]
    ssems = scratch[2 * _NBUF:4 * _NBUF]
    cid = lax.axis_index("c")
    e0 = cid * _EPS
    _ring_copy(data_hbm, fresh_hbm, out_hbm, e0, _EPS,
               sbufs, ssems[:_NBUF], ssems[_NBUF:])


def _tec_body(data_hbm, fresh_hbm, out_hbm, *scratch):
    tbufs = scratch[_NBUF:2 * _NBUF]
    tsems = scratch[4 * _NBUF:]
    wid = lax.axis_index("s") * _NUM_CORES + lax.axis_index("c")
    e0 = _ENVS_SCS + wid * _EPT
    _ring_copy(data_hbm, fresh_hbm, out_hbm, e0, _EPT,
               tbufs, tsems[:_NBUF], tsems[_NBUF:])


def kernel(data, fresh_data):
    scs_mesh = plsc.ScalarSubcoreMesh(axis_name="c")
    tec_mesh = plsc.VectorSubcoreMesh(core_axis_name="c", subcore_axis_name="s")
    tec_vmem = pltpu.MemorySpace.VMEM @ tec_mesh
    run = mpmd.mpmd_map(
        [(scs_mesh, _scs_body), (tec_mesh, _tec_body)],
        out_types=jax.ShapeDtypeStruct((_NUM_ENVS, _NUM_STEPS * _FEAT), jnp.float32),
        scratch_types=(
            [pltpu.VMEM_SHARED((_EPS, _FEAT), jnp.float32) for _ in range(_NBUF)]
            + [tec_vmem((_EPT, _FEAT), jnp.float32) for _ in range(_NBUF)]
            + [pltpu.SemaphoreType.DMA @ scs_mesh for _ in range(2 * _NBUF)]
            + [pltpu.SemaphoreType.DMA @ tec_mesh for _ in range(2 * _NBUF)]
        ),
    )
    return run(data, fresh_data)
